# Initial kernel scaffold; baseline (speedup 1.0000x reference)
import jax
import jax.numpy as jnp
from jax.experimental import pallas as pl


def kernel(x, pos, edge_index, batch, We1, We2, Wm1, Wm2, Wu1, Wu2, Wp1, Wp2, Wq1, Wq2):
    raise NotImplementedError("write your pallas kernel here")


# speedup vs baseline: 2.6592x; 2.6592x over previous
import functools

import jax
import jax.numpy as jnp
from jax import lax
from jax.experimental import pallas as pl
from jax.experimental.pallas import tpu as pltpu
from jax.experimental.pallas import tpu_sc as plsc

N_NODES = 10000
N_EDGES = 320000
NUM_CLASSES = 16
HIDDEN = 128
ATTR = 4
N_LAYERS = 4
NUM_GRAPHS = 64
OUT = 1

NC, NS = 2, 16
NW = NC * NS
CHUNK = 128
N_CHUNKS = N_EDGES // CHUNK
WORK_ITERS = (N_CHUNKS + NW - 1) // NW

_PREC = lax.Precision.HIGHEST
_F32 = jnp.float32


def _dot(a, b):
    return jnp.dot(a, b, precision=_PREC, preferred_element_type=_F32)


def _silu(v):
    return v * (1.0 / (1.0 + jnp.exp(-v)))


@functools.cache
def _make_sc_gather():
    D = HIDDEN
    mesh = plsc.VectorSubcoreMesh(core_axis_name="c", subcore_axis_name="s",
                                  num_cores=NC, num_subcores=NS)

    @functools.partial(
        pl.kernel,
        out_type=[
            jax.ShapeDtypeStruct((N_EDGES, D), _F32),
            jax.ShapeDtypeStruct((N_EDGES, D), _F32),
        ],
        mesh=mesh,
        scratch_types=[
            pltpu.VMEM((CHUNK,), jnp.int32),
            pltpu.VMEM((CHUNK,), jnp.int32),
            pltpu.VMEM((CHUNK, D), _F32),
            pltpu.VMEM((CHUNK, D), _F32),
            pltpu.SemaphoreType.DMA,
            pltpu.SemaphoreType.DMA,
        ],
    )
    def gather_k(dst_hbm, src_hbm, tab_hbm, outd_hbm, outs_hbm,
                 idx_d, idx_s, rows_d, rows_s, sem_d, sem_s):
        wid = lax.axis_index("s") * NC + lax.axis_index("c")

        def step(t, carry):
            chunk = wid + t * NW

            @pl.when(chunk < N_CHUNKS)
            def _():
                base = chunk * CHUNK
                pltpu.sync_copy(dst_hbm.at[pl.ds(base, CHUNK)], idx_d)
                pltpu.sync_copy(src_hbm.at[pl.ds(base, CHUNK)], idx_s)
                cpd = pltpu.async_copy(tab_hbm.at[idx_d], rows_d, sem_d)
                cps = pltpu.async_copy(tab_hbm.at[idx_s], rows_s, sem_s)
                cpd.wait()
                cps.wait()
                pltpu.sync_copy(rows_d, outd_hbm.at[pl.ds(base, CHUNK)])
                pltpu.sync_copy(rows_s, outs_hbm.at[pl.ds(base, CHUNK)])

            return carry

        lax.fori_loop(0, WORK_ITERS, step, 0)

    return gather_k


@functools.cache
def _make_sc_scatter():
    D = HIDDEN
    mesh = plsc.VectorSubcoreMesh(core_axis_name="c", subcore_axis_name="s",
                                  num_cores=NC, num_subcores=NS)
    rows_per = 640
    last_rows = N_NODES - rows_per * (NS - 1)

    @functools.partial(
        pl.kernel,
        out_type=jax.ShapeDtypeStruct((NC, N_NODES, D), _F32),
        mesh=mesh,
        scratch_types=[
            pltpu.VMEM((1, CHUNK), jnp.int32),
            pltpu.VMEM((CHUNK, D), _F32),
            pltpu.VMEM_SHARED((N_NODES, D), _F32),
        ],
    )
    def scatter_k(dst_hbm, vals_hbm, zeros_hbm, out_hbm, idx_v, rows_v, acc):
        cid = lax.axis_index("c")
        sid = lax.axis_index("s")
        wid = sid * NC + cid

        @pl.when(sid < NS - 1)
        def _():
            pltpu.sync_copy(zeros_hbm.at[pl.ds(sid * rows_per, rows_per)],
                            acc.at[pl.ds(sid * rows_per, rows_per)])

        @pl.when(sid == NS - 1)
        def _():
            pltpu.sync_copy(zeros_hbm.at[pl.ds(sid * rows_per, last_rows)],
                            acc.at[pl.ds(sid * rows_per, last_rows)])

        plsc.subcore_barrier()

        def step(t, carry):
            chunk = wid + t * NW

            @pl.when(chunk < N_CHUNKS)
            def _():
                base = chunk * CHUNK
                pltpu.sync_copy(dst_hbm.at[pl.ds(base, CHUNK)], idx_v.at[0])
                pltpu.sync_copy(vals_hbm.at[pl.ds(base, CHUNK)], rows_v)
                pltpu.sync_copy(rows_v, acc.at[idx_v.at[0]], add=True)

            return carry

        lax.fori_loop(0, WORK_ITERS, step, 0)
        plsc.subcore_barrier()

        @pl.when(sid < NS - 1)
        def _():
            pltpu.sync_copy(acc.at[pl.ds(sid * rows_per, rows_per)],
                            out_hbm.at[cid, pl.ds(sid * rows_per, rows_per)])

        @pl.when(sid == NS - 1)
        def _():
            pltpu.sync_copy(acc.at[pl.ds(sid * rows_per, last_rows)],
                            out_hbm.at[cid, pl.ds(sid * rows_per, last_rows)])

    return scatter_k


_RE = 1600
_GE = N_EDGES // _RE
_RN = 2000
_GN = N_NODES // _RN


def _edge_attr_body(pd_ref, ps_ref, full_ref, ea_ref):
    pd = pd_ref[...]
    ps = ps_ref[...]
    rel = ps[:, 0:3] - pd[:, 0:3]
    r2 = jnp.sum(rel * rel, axis=1, keepdims=True)
    r = jnp.sqrt(r2 + 1e-12)
    u = rel / r
    s3 = jnp.float32(3.0) ** 0.5
    one = jnp.ones_like(r2)
    ea16 = jnp.concatenate(
        [one, s3 * u[:, 1:2], s3 * u[:, 2:3], s3 * u[:, 0:1], r2, one,
         jnp.zeros((pd.shape[0], 10), _F32)], axis=1)
    ea_ref[...] = ea16
    full_ref[...] = jnp.concatenate(
        [ea16, jnp.zeros((pd.shape[0], HIDDEN - 16), _F32)], axis=1)


def _tc_edge_attr(posd, poss):
    return pl.pallas_call(
        _edge_attr_body,
        grid=(_GE,),
        in_specs=[pl.BlockSpec((_RE, HIDDEN), lambda i: (i, 0)),
                  pl.BlockSpec((_RE, HIDDEN), lambda i: (i, 0))],
        out_specs=[pl.BlockSpec((_RE, HIDDEN), lambda i: (i, 0)),
                   pl.BlockSpec((_RE, 16), lambda i: (i, 0))],
        out_shape=[jax.ShapeDtypeStruct((N_EDGES, HIDDEN), _F32),
                   jax.ShapeDtypeStruct((N_EDGES, 16), _F32)],
    )(posd, poss)


def _combine(z, attr):
    acc = attr[:, 0:1] * z[:, 0:HIDDEN]
    for a in range(1, ATTR):
        acc = acc + attr[:, a:a + 1] * z[:, a * HIDDEN:(a + 1) * HIDDEN]
    return acc


def _embed_body(x_ref, acc_ref, We1_ref, We2_ref, h_ref, na_ref):
    xv = x_ref[...]
    a = acc_ref[...]
    s = a[0] + a[1]
    cnt = jnp.maximum(s[:, 5:6], 1.0)
    na_raw = s[:, 0:4] / cnt
    na = jnp.concatenate([jnp.ones_like(cnt), na_raw[:, 1:4]], axis=1)
    onehot = (xv == lax.broadcasted_iota(jnp.int32, (xv.shape[0], NUM_CLASSES), 1)
              ).astype(_F32)
    z0 = _dot(onehot, We1_ref[...])
    u = _silu(_combine(z0, na))
    z1 = _dot(u, We2_ref[...])
    h_ref[...] = _combine(z1, na)
    na_ref[...] = na


def _tc_embed(x2d, acc, We1c, We2c):
    return pl.pallas_call(
        _embed_body,
        grid=(_GN,),
        in_specs=[pl.BlockSpec((_RN, 1), lambda i: (i, 0)),
                  pl.BlockSpec((NC, _RN, HIDDEN), lambda i: (0, i, 0)),
                  pl.BlockSpec((NUM_CLASSES, ATTR * HIDDEN), lambda i: (0, 0)),
                  pl.BlockSpec((HIDDEN, ATTR * HIDDEN), lambda i: (0, 0))],
        out_specs=[pl.BlockSpec((_RN, HIDDEN), lambda i: (i, 0)),
                   pl.BlockSpec((_RN, 4), lambda i: (i, 0))],
        out_shape=[jax.ShapeDtypeStruct((N_NODES, HIDDEN), _F32),
                   jax.ShapeDtypeStruct((N_NODES, 4), _F32)],
    )(x2d, acc, We1c, We2c)


def _edge_mlp_body(hd_ref, hs_ref, ea_ref, A_ref, B_ref, c_ref, W2_ref, out_ref):
    ea = ea_ref[...]
    z = _dot(hd_ref[...], A_ref[...]) + _dot(hs_ref[...], B_ref[...])
    z = z + ea[:, 4:5] * c_ref[...]
    m1 = _silu(_combine(z, ea))
    z2 = _dot(m1, W2_ref[...])
    out_ref[...] = _silu(_combine(z2, ea))


def _tc_edge_mlp(hd, hs, ea, A, B, cvec, W2):
    return pl.pallas_call(
        _edge_mlp_body,
        grid=(_GE,),
        in_specs=[pl.BlockSpec((_RE, HIDDEN), lambda i: (i, 0)),
                  pl.BlockSpec((_RE, HIDDEN), lambda i: (i, 0)),
                  pl.BlockSpec((_RE, 16), lambda i: (i, 0)),
                  pl.BlockSpec((HIDDEN, ATTR * HIDDEN), lambda i: (0, 0)),
                  pl.BlockSpec((HIDDEN, ATTR * HIDDEN), lambda i: (0, 0)),
                  pl.BlockSpec((1, ATTR * HIDDEN), lambda i: (0, 0)),
                  pl.BlockSpec((HIDDEN, ATTR * HIDDEN), lambda i: (0, 0))],
        out_specs=pl.BlockSpec((_RE, HIDDEN), lambda i: (i, 0)),
        out_shape=jax.ShapeDtypeStruct((N_EDGES, HIDDEN), _F32),
    )(hd, hs, ea, A, B, cvec, W2)


def _update_body(h_ref, agg_ref, na_ref, Wu1_ref, Wu2_ref, out_ref):
    h = h_ref[...]
    a = agg_ref[...]
    agg = a[0] + a[1]
    na = na_ref[...]
    cat = jnp.concatenate([h, agg], axis=1)
    z = _dot(cat, Wu1_ref[...])
    u = _silu(_combine(z, na))
    z2 = _dot(u, Wu2_ref[...])
    out_ref[...] = h + _combine(z2, na)


def _tc_update(h, agg, na, Wu1c, Wu2c):
    return pl.pallas_call(
        _update_body,
        grid=(_GN,),
        in_specs=[pl.BlockSpec((_RN, HIDDEN), lambda i: (i, 0)),
                  pl.BlockSpec((NC, _RN, HIDDEN), lambda i: (0, i, 0)),
                  pl.BlockSpec((_RN, 4), lambda i: (i, 0)),
                  pl.BlockSpec((2 * HIDDEN, ATTR * HIDDEN), lambda i: (0, 0)),
                  pl.BlockSpec((HIDDEN, ATTR * HIDDEN), lambda i: (0, 0))],
        out_specs=pl.BlockSpec((_RN, HIDDEN), lambda i: (i, 0)),
        out_shape=jax.ShapeDtypeStruct((N_NODES, HIDDEN), _F32),
    )(h, agg, na, Wu1c, Wu2c)


def _head_body(h_ref, na_ref, b_ref, Wp1_ref, Wp2_ref, Wq1_ref, Wq2_ref,
               out_ref, pooled, cnt):
    i = pl.program_id(0)

    @pl.when(i == 0)
    def _():
        pooled[...] = jnp.zeros_like(pooled)
        cnt[...] = jnp.zeros_like(cnt)

    z = _dot(h_ref[...], Wp1_ref[...])
    hp = _silu(_combine(z, na_ref[...]))
    hp2 = _dot(hp, Wp2_ref[...])
    oneB = (b_ref[...] == lax.broadcasted_iota(jnp.int32, (_RN, NUM_GRAPHS), 1)
            ).astype(_F32)
    pooled[...] += lax.dot_general(oneB, hp2, (((0,), (0,)), ((), ())),
                                   precision=_PREC, preferred_element_type=_F32)
    cnt[...] += lax.dot_general(oneB, jnp.ones((_RN, 1), _F32),
                                (((0,), (0,)), ((), ())),
                                precision=_PREC, preferred_element_type=_F32)

    @pl.when(i == _GN - 1)
    def _():
        pm = pooled[...] / jnp.maximum(cnt[...], 1.0)
        t = _silu(_dot(pm, Wq1_ref[...]))
        out_ref[...] = _dot(t, Wq2_ref[...])


def _tc_head(h, na, batch2d, Wp1c, Wp2, Wq1, Wq2):
    return pl.pallas_call(
        _head_body,
        grid=(_GN,),
        in_specs=[pl.BlockSpec((_RN, HIDDEN), lambda i: (i, 0)),
                  pl.BlockSpec((_RN, 4), lambda i: (i, 0)),
                  pl.BlockSpec((_RN, 1), lambda i: (i, 0)),
                  pl.BlockSpec((HIDDEN, ATTR * HIDDEN), lambda i: (0, 0)),
                  pl.BlockSpec((HIDDEN, HIDDEN), lambda i: (0, 0)),
                  pl.BlockSpec((HIDDEN, HIDDEN), lambda i: (0, 0)),
                  pl.BlockSpec((HIDDEN, OUT), lambda i: (0, 0))],
        out_specs=pl.BlockSpec((NUM_GRAPHS, OUT), lambda i: (0, 0)),
        out_shape=jax.ShapeDtypeStruct((NUM_GRAPHS, OUT), _F32),
        scratch_shapes=[pltpu.VMEM((NUM_GRAPHS, HIDDEN), _F32),
                        pltpu.VMEM((NUM_GRAPHS, 1), _F32)],
    )(h, na, batch2d, Wp1c, Wp2, Wq1, Wq2)


def kernel(x, pos, edge_index, batch, We1, We2, Wm1, Wm2, Wu1, Wu2,
           Wp1, Wp2, Wq1, Wq2):
    src = edge_index[0]
    dst = edge_index[1]
    pos_pad = jnp.pad(pos.astype(_F32), ((0, 0), (0, HIDDEN - 3)))
    x2d = x[:, None]
    batch2d = batch[:, None]

    zeros128 = jnp.zeros((N_NODES, HIDDEN), _F32)

    We1c = We1.reshape(NUM_CLASSES, ATTR * HIDDEN)
    We2c = We2.reshape(HIDDEN, ATTR * HIDDEN)
    Wp1c = Wp1.reshape(HIDDEN, ATTR * HIDDEN)

    posd, poss = _make_sc_gather()(dst, src, pos_pad)
    ea_full, ea = _tc_edge_attr(posd, poss)
    acc = _make_sc_scatter()(dst, ea_full, zeros128)
    h, na = _tc_embed(x2d, acc, We1c, We2c)

    for i in range(N_LAYERS):
        Wr = Wm1[i].reshape(2 * HIDDEN + 1, ATTR, HIDDEN)
        A = Wr[:HIDDEN].reshape(HIDDEN, ATTR * HIDDEN)
        B = Wr[HIDDEN:2 * HIDDEN].reshape(HIDDEN, ATTR * HIDDEN)
        cvec = Wr[2 * HIDDEN].reshape(1, ATTR * HIDDEN)
        W2 = Wm2[i].reshape(HIDDEN, ATTR * HIDDEN)
        Wu1c = Wu1[i].reshape(2 * HIDDEN, ATTR * HIDDEN)
        Wu2c = Wu2[i].reshape(HIDDEN, ATTR * HIDDEN)

        hd, hs = _make_sc_gather()(dst, src, h)
        m2 = _tc_edge_mlp(hd, hs, ea, A, B, cvec, W2)
        agg = _make_sc_scatter()(dst, m2, zeros128)
        h = _tc_update(h, agg, na, Wu1c, Wu2c)

    return _tc_head(h, na, batch2d, Wp1c, Wp2, Wq1, Wq2)
